# baseline (device time: 20573 ns/iter reference)
import jax
import jax.numpy as jnp
from jax import lax
from jax.experimental import pallas as pl
from jax.experimental.pallas import tpu as pltpu

N_DEV = 4
N_EXPERTS = 16
N_LOCAL_E = 4
K_CAP = 128


def _tdot(a, b):
    return lax.dot_general(a, b, (((0,), (0,)), ((), ())),
                           preferred_element_type=jnp.float32)


def kernel(x, router_W, route_idx, expert_W, shared_W):
    n, d = x.shape
    _, _, h = expert_W.shape
    chunk = n // N_DEV

    def body(x_ref, router_ref, idx_ref, expW_ref, sharedW_ref,
             out_ref, coeff_ref, y_send, e_send, y_recv, e_recv,
             send_y_sems, send_e_sems, recv_y_sems, recv_e_sems):
        my_pos = lax.axis_index("i")

        barrier_sem = pltpu.get_barrier_semaphore()
        for dd in range(1, N_DEV):
            pl.semaphore_signal(
                barrier_sem, inc=1,
                device_id=(lax.rem(my_pos + dd, N_DEV),),
                device_id_type=pl.DeviceIdType.MESH,
            )

        xv = x_ref[:, :]
        scores = jnp.dot(xv, router_ref[:, :],
                         preferred_element_type=jnp.float32,
                         precision=lax.Precision.HIGHEST)
        scores = scores - jnp.max(scores, axis=-1, keepdims=True)
        ex = jnp.exp(scores)
        probs = ex / jnp.sum(ex, axis=-1, keepdims=True)
        lanes = lax.broadcasted_iota(jnp.int32, (n, N_EXPERTS), 1)
        gate = probs * (lanes == idx_ref[:, :]).astype(jnp.float32)
        r16 = lax.broadcasted_iota(jnp.int32, (N_EXPERTS, N_LOCAL_E), 0)
        c4 = lax.broadcasted_iota(jnp.int32, (N_EXPERTS, N_LOCAL_E), 1)
        sel = (r16 == N_LOCAL_E * my_pos + c4).astype(jnp.float32)
        coeff_ref[:, :] = jnp.dot(gate, sel, preferred_element_type=jnp.float32)

        tri_r = lax.broadcasted_iota(jnp.int32, (chunk, chunk), 0)
        tri_c = lax.broadcasted_iota(jnp.int32, (chunk, chunk), 1)
        t_strict = (tri_c < tri_r).astype(jnp.float32)
        klane = lax.broadcasted_iota(jnp.int32, (chunk, K_CAP), 1)

        def compact(c):
            cf = coeff_ref[pl.ds(c * chunk, chunk), :]
            mask = (jnp.sum(cf, axis=1, keepdims=True) > 0.0)
            maskf = mask.astype(jnp.float32)
            rank = jnp.dot(t_strict, maskf,
                           preferred_element_type=jnp.float32)
            rank_i = rank.astype(jnp.int32)
            e_mat = jnp.where(mask, (klane == rank_i).astype(jnp.float32), 0.0)
            return e_mat, cf

        def sparse_y(c, e_mat, cf):
            xg = _tdot(e_mat, x_ref[pl.ds(c * chunk, chunk), :])
            cfg = _tdot(e_mat, cf)
            acc = jnp.zeros((K_CAP, h), jnp.float32)
            for el in range(N_LOCAL_E):
                y = jnp.dot(xg, expW_ref[el], preferred_element_type=jnp.float32)
                acc = acc + cfg[:, el:el + 1] * y
            return acc

        rdmas = {}
        for dd in (2, 1, 3):
            slot = dd - 1
            tgt = lax.rem(my_pos + dd, N_DEV)
            c = tgt
            e_mat, cf = compact(c)
            y_send[slot, :, :] = sparse_y(c, e_mat, cf).astype(jnp.bfloat16)
            e_send[slot, :, :] = e_mat.astype(jnp.bfloat16)
            if dd == 2:
                pl.semaphore_wait(barrier_sem, N_DEV - 1)
            rdma_y = pltpu.make_async_remote_copy(
                src_ref=y_send.at[slot], dst_ref=y_recv.at[slot],
                send_sem=send_y_sems.at[slot], recv_sem=recv_y_sems.at[slot],
                device_id=(tgt,), device_id_type=pl.DeviceIdType.MESH,
            )
            rdma_e = pltpu.make_async_remote_copy(
                src_ref=e_send.at[slot], dst_ref=e_recv.at[slot],
                send_sem=send_e_sems.at[slot], recv_sem=recv_e_sems.at[slot],
                device_id=(tgt,), device_id_type=pl.DeviceIdType.MESH,
            )
            rdma_y.start()
            rdma_e.start()
            rdmas[dd] = (rdma_y, rdma_e)

        e_own, cf_own = compact(my_pos)
        y_own = sparse_y(my_pos, e_own, cf_own)
        own = jnp.dot(e_own, y_own, preferred_element_type=jnp.float32)
        shared_chunk = jnp.dot(
            x_ref[pl.ds(my_pos * chunk, chunk), :], sharedW_ref[:, :],
            preferred_element_type=jnp.float32,
        )
        acc = own + shared_chunk

        for dd in (2, 1, 3):
            rdma_y, rdma_e = rdmas[dd]
            rdma_y.wait()
            rdma_e.wait()
            slot = dd - 1
            acc = acc + jnp.dot(
                e_recv[slot, :, :], y_recv[slot, :, :],
                preferred_element_type=jnp.float32,
            )
        out_ref[:, :] = acc

    return pl.pallas_call(
        body,
        out_shape=jax.ShapeDtypeStruct((chunk, h), jnp.float32),
        in_specs=[pl.BlockSpec(memory_space=pltpu.VMEM)] * 5,
        out_specs=pl.BlockSpec(memory_space=pltpu.VMEM),
        scratch_shapes=[
            pltpu.VMEM((n, N_LOCAL_E), jnp.float32),
            pltpu.VMEM((3, K_CAP, h), jnp.bfloat16),
            pltpu.VMEM((3, chunk, K_CAP), jnp.bfloat16),
            pltpu.VMEM((3, K_CAP, h), jnp.bfloat16),
            pltpu.VMEM((3, chunk, K_CAP), jnp.bfloat16),
            pltpu.SemaphoreType.DMA((3,)),
            pltpu.SemaphoreType.DMA((3,)),
            pltpu.SemaphoreType.DMA((3,)),
            pltpu.SemaphoreType.DMA((3,)),
        ],
        compiler_params=pltpu.CompilerParams(collective_id=0),
    )(x, router_W, route_idx, expert_W, shared_W)


# device time: 16995 ns/iter; 1.2105x vs baseline; 1.2105x over previous
import jax
import jax.numpy as jnp
from jax import lax
from jax.experimental import pallas as pl
from jax.experimental.pallas import tpu as pltpu

N_DEV = 4
N_EXPERTS = 16
N_LOCAL_E = 4
CAP = 48


def _tdot(a, b):
    return lax.dot_general(a, b, (((0,), (0,)), ((), ())),
                           preferred_element_type=jnp.float32)


def kernel(x, router_W, route_idx, expert_W, shared_W):
    n, d = x.shape
    _, _, h = expert_W.shape
    chunk = n // N_DEV
    grp = N_LOCAL_E * CAP

    def body(x_ref, router_ref, idx_ref, expW_ref, sharedW_ref,
             out_ref, coeff_ref, y_send, y_recv, send_sems, recv_sems):
        my_pos = lax.axis_index("i")

        barrier_sem = pltpu.get_barrier_semaphore()
        for dd in range(1, N_DEV):
            pl.semaphore_signal(
                barrier_sem, inc=1,
                device_id=(lax.rem(my_pos + dd, N_DEV),),
                device_id_type=pl.DeviceIdType.MESH,
            )

        xv = x_ref[:, :]
        scores = jnp.dot(xv, router_ref[:, :],
                         preferred_element_type=jnp.float32,
                         precision=lax.Precision.HIGHEST)
        scores = scores - jnp.max(scores, axis=-1, keepdims=True)
        ex = jnp.exp(scores)
        probs = ex / jnp.sum(ex, axis=-1, keepdims=True)
        lanes = lax.broadcasted_iota(jnp.int32, (n, N_EXPERTS), 1)
        gate = probs * (lanes == idx_ref[:, :]).astype(jnp.float32)
        r16 = lax.broadcasted_iota(jnp.int32, (N_EXPERTS, N_LOCAL_E), 0)
        c4 = lax.broadcasted_iota(jnp.int32, (N_EXPERTS, N_LOCAL_E), 1)
        sel = (r16 == N_LOCAL_E * my_pos + c4).astype(jnp.float32)
        coeff_ref[:, :] = jnp.dot(gate, sel, preferred_element_type=jnp.float32)

        tri_r = lax.broadcasted_iota(jnp.int32, (chunk, chunk), 0)
        tri_c = lax.broadcasted_iota(jnp.int32, (chunk, chunk), 1)
        t_strict = (tri_c < tri_r).astype(jnp.float32)
        cap_lane = lax.broadcasted_iota(jnp.int32, (chunk, CAP), 1)
        el_lane = lax.broadcasted_iota(jnp.int32, (chunk, N_LOCAL_E), 1)

        def build_E(row0, owner):
            idx_c = idx_ref[pl.ds(row0, chunk), :]
            masksf = (idx_c == N_LOCAL_E * owner + el_lane).astype(
                jnp.float32)
            ranks = jnp.dot(t_strict, masksf,
                            preferred_element_type=jnp.float32)
            ranks_i = ranks.astype(jnp.int32)
            blocks = []
            for el in range(N_LOCAL_E):
                hit = (cap_lane == ranks_i[:, el:el + 1]).astype(
                    jnp.float32)
                blocks.append(hit * masksf[:, el:el + 1])
            return jnp.concatenate(blocks, axis=1)

        e_mats, xgs, cfgs = [], [], []
        for o in range(N_DEV):
            c = lax.rem(my_pos + o, N_DEV)
            row0 = c * chunk
            e_mat = build_E(row0, my_pos)
            e_mats.append(e_mat)
            xgs.append(_tdot(e_mat, x_ref[pl.ds(row0, chunk), :]))
            cfgs.append(_tdot(e_mat, coeff_ref[pl.ds(row0, chunk), :]))

        rdmas = []
        own_blocks = []
        for el in range(N_LOCAL_E):
            xga = jnp.concatenate(
                [xgs[o][el * CAP:(el + 1) * CAP, :] for o in range(N_DEV)],
                axis=0)
            cf_col = jnp.concatenate(
                [cfgs[o][el * CAP:(el + 1) * CAP, el:el + 1]
                 for o in range(N_DEV)], axis=0)
            y_el = jnp.dot(xga, expW_ref[el],
                           preferred_element_type=jnp.float32) * cf_col
            own_blocks.append(y_el[0:CAP, :])
            for o in range(1, N_DEV):
                y_send[o - 1, el, :, :] = (
                    y_el[o * CAP:(o + 1) * CAP, :].astype(jnp.bfloat16))
            if el == 0:
                pl.semaphore_wait(barrier_sem, N_DEV - 1)
            for o in range(1, N_DEV):
                tgt = lax.rem(my_pos + o, N_DEV)
                rdma = pltpu.make_async_remote_copy(
                    src_ref=y_send.at[o - 1, el],
                    dst_ref=y_recv.at[o - 1, el],
                    send_sem=send_sems.at[o - 1, el],
                    recv_sem=recv_sems.at[o - 1, el],
                    device_id=(tgt,), device_id_type=pl.DeviceIdType.MESH,
                )
                rdma.start()
                rdmas.append(rdma)

        own_y = jnp.concatenate(own_blocks, axis=0)
        acc = jnp.dot(e_mats[0], own_y, preferred_element_type=jnp.float32)
        acc = acc + jnp.dot(
            x_ref[pl.ds(my_pos * chunk, chunk), :], sharedW_ref[:, :],
            preferred_element_type=jnp.float32,
        )

        for rdma in rdmas:
            rdma.wait()
        for j in range(N_DEV - 1):
            src = lax.rem(my_pos + N_DEV - (j + 1), N_DEV)
            e_src = build_E(my_pos * chunk, src)
            y_j = y_recv[j, :, :, :].reshape(grp, h).astype(jnp.float32)
            acc = acc + jnp.dot(e_src, y_j,
                                preferred_element_type=jnp.float32)
        out_ref[:, :] = acc

    return pl.pallas_call(
        body,
        out_shape=jax.ShapeDtypeStruct((chunk, h), jnp.float32),
        in_specs=[pl.BlockSpec(memory_space=pltpu.VMEM)] * 5,
        out_specs=pl.BlockSpec(memory_space=pltpu.VMEM),
        scratch_shapes=[
            pltpu.VMEM((n, N_LOCAL_E), jnp.float32),
            pltpu.VMEM((3, N_LOCAL_E, CAP, h), jnp.bfloat16),
            pltpu.VMEM((3, N_LOCAL_E, CAP, h), jnp.bfloat16),
            pltpu.SemaphoreType.DMA((3, N_LOCAL_E)),
            pltpu.SemaphoreType.DMA((3, N_LOCAL_E)),
        ],
        compiler_params=pltpu.CompilerParams(collective_id=0),
    )(x, router_W, route_idx, expert_W, shared_W)


# device time: 9622 ns/iter; 2.1381x vs baseline; 1.7663x over previous
import jax
import jax.numpy as jnp
from jax import lax
from jax.experimental import pallas as pl
from jax.experimental.pallas import tpu as pltpu

N_DEV = 4
N_EXPERTS = 16
N_LOCAL_E = 4
CAP = 48


def _tdot(a, b):
    return lax.dot_general(a, b, (((0,), (0,)), ((), ())),
                           preferred_element_type=jnp.float32)


def kernel(x, router_W, route_idx, expert_W, shared_W):
    n, d = x.shape
    _, _, h = expert_W.shape
    chunk = n // N_DEV
    grp = N_LOCAL_E * CAP

    def body(x_ref, router_ref, idx_ref, expW_hbm, sharedW_hbm,
             out_ref, coeff_ref, w_vmem, sharedw_vmem, y_send, y_recv,
             send_sems, recv_sems, wload_sems, sload_sem):
        my_pos = lax.axis_index("i")

        w_copies = []
        for el in range(N_LOCAL_E):
            cp = pltpu.make_async_copy(
                expW_hbm.at[el], w_vmem.at[el], wload_sems.at[el])
            cp.start()
            w_copies.append(cp)
        s_copy = pltpu.make_async_copy(sharedW_hbm, sharedw_vmem, sload_sem)
        s_copy.start()

        barrier_sem = pltpu.get_barrier_semaphore()
        for dd in range(1, N_DEV):
            pl.semaphore_signal(
                barrier_sem, inc=1,
                device_id=(lax.rem(my_pos + dd, N_DEV),),
                device_id_type=pl.DeviceIdType.MESH,
            )

        xv = x_ref[:, :]
        scores = jnp.dot(xv, router_ref[:, :],
                         preferred_element_type=jnp.float32,
                         precision=lax.Precision.HIGHEST)
        scores = scores - jnp.max(scores, axis=-1, keepdims=True)
        ex = jnp.exp(scores)
        probs = ex / jnp.sum(ex, axis=-1, keepdims=True)
        lanes = lax.broadcasted_iota(jnp.int32, (n, N_EXPERTS), 1)
        gate = probs * (lanes == idx_ref[:, :]).astype(jnp.float32)
        r16 = lax.broadcasted_iota(jnp.int32, (N_EXPERTS, N_LOCAL_E), 0)
        c4 = lax.broadcasted_iota(jnp.int32, (N_EXPERTS, N_LOCAL_E), 1)
        sel = (r16 == N_LOCAL_E * my_pos + c4).astype(jnp.float32)
        coeff_ref[:, :] = jnp.dot(gate, sel, preferred_element_type=jnp.float32)

        tri_r = lax.broadcasted_iota(jnp.int32, (chunk, chunk), 0)
        tri_c = lax.broadcasted_iota(jnp.int32, (chunk, chunk), 1)
        t_strict = (tri_c < tri_r).astype(jnp.float32)
        cap_lane = lax.broadcasted_iota(jnp.int32, (chunk, CAP), 1)
        el_lane = lax.broadcasted_iota(jnp.int32, (chunk, N_LOCAL_E), 1)

        def build_E(row0, owner):
            idx_c = idx_ref[pl.ds(row0, chunk), :]
            masksf = (idx_c == N_LOCAL_E * owner + el_lane).astype(
                jnp.float32)
            ranks = jnp.dot(t_strict, masksf,
                            preferred_element_type=jnp.float32)
            ranks_i = ranks.astype(jnp.int32)
            blocks = []
            for el in range(N_LOCAL_E):
                hit = (cap_lane == ranks_i[:, el:el + 1]).astype(
                    jnp.float32)
                blocks.append(hit * masksf[:, el:el + 1])
            return jnp.concatenate(blocks, axis=1)

        e_mats, xgs, cfgs = [], [], []
        for o in range(N_DEV):
            c = lax.rem(my_pos + o, N_DEV)
            row0 = c * chunk
            e_mat = build_E(row0, my_pos)
            e_mats.append(e_mat)
            xgs.append(_tdot(e_mat, x_ref[pl.ds(row0, chunk), :]))
            cfgs.append(_tdot(e_mat, coeff_ref[pl.ds(row0, chunk), :]))

        rdmas = []
        own_blocks = []
        for el in range(N_LOCAL_E):
            xga = jnp.concatenate(
                [xgs[o][el * CAP:(el + 1) * CAP, :] for o in range(N_DEV)],
                axis=0)
            cf_col = jnp.concatenate(
                [cfgs[o][el * CAP:(el + 1) * CAP, el:el + 1]
                 for o in range(N_DEV)], axis=0)
            w_copies[el].wait()
            y_el = jnp.dot(xga, w_vmem[el],
                           preferred_element_type=jnp.float32) * cf_col
            own_blocks.append(y_el[0:CAP, :])
            for o in range(1, N_DEV):
                y_send[o - 1, el, :, :] = (
                    y_el[o * CAP:(o + 1) * CAP, :].astype(jnp.bfloat16))
            if el == 0:
                pl.semaphore_wait(barrier_sem, N_DEV - 1)
            for o in range(1, N_DEV):
                tgt = lax.rem(my_pos + o, N_DEV)
                rdma = pltpu.make_async_remote_copy(
                    src_ref=y_send.at[o - 1, el],
                    dst_ref=y_recv.at[o - 1, el],
                    send_sem=send_sems.at[o - 1, el],
                    recv_sem=recv_sems.at[o - 1, el],
                    device_id=(tgt,), device_id_type=pl.DeviceIdType.MESH,
                )
                rdma.start()
                rdmas.append(rdma)

        own_y = jnp.concatenate(own_blocks, axis=0)
        acc = jnp.dot(e_mats[0], own_y, preferred_element_type=jnp.float32)
        s_copy.wait()
        acc = acc + jnp.dot(
            x_ref[pl.ds(my_pos * chunk, chunk), :], sharedw_vmem[:, :],
            preferred_element_type=jnp.float32,
        )

        for rdma in rdmas:
            rdma.wait()
        for j in range(N_DEV - 1):
            src = lax.rem(my_pos + N_DEV - (j + 1), N_DEV)
            e_src = build_E(my_pos * chunk, src)
            y_j = y_recv[j, :, :, :].reshape(grp, h).astype(jnp.float32)
            acc = acc + jnp.dot(e_src, y_j,
                                preferred_element_type=jnp.float32)
        out_ref[:, :] = acc

    return pl.pallas_call(
        body,
        out_shape=jax.ShapeDtypeStruct((chunk, h), jnp.float32),
        in_specs=[
            pl.BlockSpec(memory_space=pltpu.VMEM),
            pl.BlockSpec(memory_space=pltpu.VMEM),
            pl.BlockSpec(memory_space=pltpu.VMEM),
            pl.BlockSpec(memory_space=pltpu.MemorySpace.HBM),
            pl.BlockSpec(memory_space=pltpu.MemorySpace.HBM),
        ],
        out_specs=pl.BlockSpec(memory_space=pltpu.VMEM),
        scratch_shapes=[
            pltpu.VMEM((n, N_LOCAL_E), jnp.float32),
            pltpu.VMEM((N_LOCAL_E, d, h), jnp.float32),
            pltpu.VMEM((d, h), jnp.float32),
            pltpu.VMEM((3, N_LOCAL_E, CAP, h), jnp.bfloat16),
            pltpu.VMEM((3, N_LOCAL_E, CAP, h), jnp.bfloat16),
            pltpu.SemaphoreType.DMA((3, N_LOCAL_E)),
            pltpu.SemaphoreType.DMA((3, N_LOCAL_E)),
            pltpu.SemaphoreType.DMA((N_LOCAL_E,)),
            pltpu.SemaphoreType.DMA,
        ],
        compiler_params=pltpu.CompilerParams(collective_id=0),
    )(x, router_W, route_idx, expert_W, shared_W)
